# baseline (device time: 22995 ns/iter reference)
import functools

import jax
import jax.numpy as jnp
from jax import lax
from jax.experimental import pallas as pl
from jax.experimental.pallas import tpu as pltpu

N_DEV = 4


def kernel(x, router_W, route_idx, expert_W, shared_W):
    n, d = x.shape
    n_exp = router_W.shape[1]
    e_per, _, h = expert_W.shape
    chunk = n // N_DEV

    def body(x_ref, rw_ref, idx_ref, ew_ref, sw_ref, out_ref,
             acc_ref, comm_ref, send_sems, recv_sems):
        my_pos = lax.axis_index("i")
        left = lax.rem(my_pos + N_DEV - 1, N_DEV)
        right = lax.rem(my_pos + 1, N_DEV)

        barrier_sem = pltpu.get_barrier_semaphore()
        for nbr in [left, right]:
            pl.semaphore_signal(
                barrier_sem, inc=1,
                device_id=(nbr,), device_id_type=pl.DeviceIdType.MESH,
            )
        pl.semaphore_wait(barrier_sem, 2)

        xv = x_ref[:, :]
        scores = jnp.dot(xv, rw_ref[:, :], preferred_element_type=jnp.float32)
        s_max = jnp.max(scores, axis=-1, keepdims=True)
        e = jnp.exp(scores - s_max)
        probs = e / jnp.sum(e, axis=-1, keepdims=True)

        cols = lax.broadcasted_iota(jnp.int32, (n, n_exp), 1)
        idx = idx_ref[:, :]

        partial = jnp.zeros((n, h), dtype=jnp.float32)
        for e_local in range(e_per):
            ge = my_pos * e_per + e_local
            p_e = jnp.sum(probs * (cols == ge).astype(jnp.float32),
                          axis=-1, keepdims=True)
            w = p_e * (idx == ge).astype(jnp.float32)
            xs = xv * w
            partial = partial + jnp.dot(
                xs, ew_ref[e_local], preferred_element_type=jnp.float32)
        acc_ref[:, :, :] = partial.reshape(N_DEV, chunk, h)

        x_mine = x_ref[pl.ds(my_pos * chunk, chunk), :]
        shared_mine = jnp.dot(x_mine, sw_ref[:, :],
                              preferred_element_type=jnp.float32)

        for hh in range(N_DEV - 1):
            send_c = lax.rem(my_pos + (N_DEV - 1 - hh), N_DEV)
            recv_c = lax.rem(my_pos + (N_DEV - 2 - hh), N_DEV)
            rdma = pltpu.make_async_remote_copy(
                src_ref=acc_ref.at[send_c],
                dst_ref=comm_ref.at[hh],
                send_sem=send_sems.at[hh],
                recv_sem=recv_sems.at[hh],
                device_id=(right,),
                device_id_type=pl.DeviceIdType.MESH,
            )
            rdma.start()
            rdma.wait()
            acc_ref[recv_c] = acc_ref[recv_c] + comm_ref[hh]

        out_ref[:, :] = acc_ref[my_pos] + shared_mine

    return pl.pallas_call(
        body,
        out_shape=jax.ShapeDtypeStruct((chunk, h), jnp.float32),
        in_specs=[pl.BlockSpec(memory_space=pltpu.VMEM)] * 5,
        out_specs=pl.BlockSpec(memory_space=pltpu.VMEM),
        scratch_shapes=[
            pltpu.VMEM((N_DEV, chunk, h), jnp.float32),
            pltpu.VMEM((N_DEV - 1, chunk, h), jnp.float32),
            pltpu.SemaphoreType.DMA((N_DEV - 1,)),
            pltpu.SemaphoreType.DMA((N_DEV - 1,)),
        ],
        compiler_params=pltpu.CompilerParams(collective_id=0),
    )(x, router_W, route_idx, expert_W, shared_W)


# device time: 16876 ns/iter; 1.3626x vs baseline; 1.3626x over previous
import functools

import jax
import jax.numpy as jnp
from jax import lax
from jax.experimental import pallas as pl
from jax.experimental.pallas import tpu as pltpu

N_DEV = 4


def kernel(x, router_W, route_idx, expert_W, shared_W):
    n, d = x.shape
    n_exp = router_W.shape[1]
    e_per, _, h = expert_W.shape
    chunk = n // N_DEV

    def body(x_ref, rw_ref, idx_ref, ew_ref, sw_ref, out_ref,
             acc_ref, comm_ref, send_sems, recv_sems):
        my_pos = lax.axis_index("i")

        barrier_sem = pltpu.get_barrier_semaphore()
        for o in range(1, N_DEV):
            pl.semaphore_signal(
                barrier_sem, inc=1,
                device_id=(lax.rem(my_pos + o, N_DEV),),
                device_id_type=pl.DeviceIdType.MESH,
            )
        pl.semaphore_wait(barrier_sem, N_DEV - 1)

        xv = x_ref[:, :]
        scores = jnp.dot(xv, rw_ref[:, :], preferred_element_type=jnp.float32)
        s_max = jnp.max(scores, axis=-1, keepdims=True)
        e = jnp.exp(scores - s_max)
        probs = e / jnp.sum(e, axis=-1, keepdims=True)

        cols = lax.broadcasted_iota(jnp.int32, (n, n_exp), 1)
        idx = idx_ref[:, :]

        partial = jnp.zeros((n, h), dtype=jnp.float32)
        for e_local in range(e_per):
            ge = my_pos * e_per + e_local
            p_e = jnp.sum(probs * (cols == ge).astype(jnp.float32),
                          axis=-1, keepdims=True)
            w = p_e * (idx == ge).astype(jnp.float32)
            xs = xv * w
            partial = partial + jnp.dot(
                xs, ew_ref[e_local], preferred_element_type=jnp.float32)
        acc_ref[:, :, :] = partial.reshape(N_DEV, chunk, h)

        rdmas = []
        for o in range(1, N_DEV):
            q = lax.rem(my_pos + o, N_DEV)
            rdma = pltpu.make_async_remote_copy(
                src_ref=acc_ref.at[q],
                dst_ref=comm_ref.at[o - 1],
                send_sem=send_sems.at[o - 1],
                recv_sem=recv_sems.at[o - 1],
                device_id=(q,),
                device_id_type=pl.DeviceIdType.MESH,
            )
            rdma.start()
            rdmas.append(rdma)

        x_mine = x_ref[pl.ds(my_pos * chunk, chunk), :]
        shared_mine = jnp.dot(x_mine, sw_ref[:, :],
                              preferred_element_type=jnp.float32)

        for rdma in rdmas:
            rdma.wait_recv()
        out_ref[:, :] = (acc_ref[my_pos] + shared_mine
                         + comm_ref[0] + comm_ref[1] + comm_ref[2])
        for rdma in rdmas:
            rdma.wait_send()

    return pl.pallas_call(
        body,
        out_shape=jax.ShapeDtypeStruct((chunk, h), jnp.float32),
        in_specs=[pl.BlockSpec(memory_space=pltpu.VMEM)] * 5,
        out_specs=pl.BlockSpec(memory_space=pltpu.VMEM),
        scratch_shapes=[
            pltpu.VMEM((N_DEV, chunk, h), jnp.float32),
            pltpu.VMEM((N_DEV - 1, chunk, h), jnp.float32),
            pltpu.SemaphoreType.DMA((N_DEV - 1,)),
            pltpu.SemaphoreType.DMA((N_DEV - 1,)),
        ],
        compiler_params=pltpu.CompilerParams(collective_id=0),
    )(x, router_W, route_idx, expert_W, shared_W)


# device time: 14170 ns/iter; 1.6228x vs baseline; 1.1910x over previous
import functools

import jax
import jax.numpy as jnp
from jax import lax
from jax.experimental import pallas as pl
from jax.experimental.pallas import tpu as pltpu

N_DEV = 4


def kernel(x, router_W, route_idx, expert_W, shared_W):
    n, d = x.shape
    n_exp = router_W.shape[1]
    e_per, _, h = expert_W.shape
    chunk = n // N_DEV

    def body(x_ref, rw_ref, idx_ref, ew_ref, sw_ref, out_ref,
             acc_ref, comm_ref, send_sems, recv_sems):
        my_pos = lax.axis_index("i")

        barrier_sem = pltpu.get_barrier_semaphore()
        for o in range(1, N_DEV):
            pl.semaphore_signal(
                barrier_sem, inc=1,
                device_id=(lax.rem(my_pos + o, N_DEV),),
                device_id_type=pl.DeviceIdType.MESH,
            )
        pl.semaphore_wait(barrier_sem, N_DEV - 1)

        xv = x_ref[:, :]
        scores = jnp.dot(xv, rw_ref[:, :], preferred_element_type=jnp.float32)
        s_max = jnp.max(scores, axis=-1, keepdims=True)
        e = jnp.exp(scores - s_max)
        probs = e / jnp.sum(e, axis=-1, keepdims=True)

        cols = lax.broadcasted_iota(jnp.int32, (n, n_exp), 1)
        idx = idx_ref[:, :]

        partial = jnp.zeros((n, h), dtype=jnp.float32)
        for e_local in range(e_per):
            ge = my_pos * e_per + e_local
            p_e = jnp.sum(probs * (cols == ge).astype(jnp.float32),
                          axis=-1, keepdims=True)
            w = p_e * (idx == ge).astype(jnp.float32)
            xs = (xv * w).astype(jnp.bfloat16)
            partial = partial + jnp.dot(
                xs, ew_ref[e_local].astype(jnp.bfloat16),
                preferred_element_type=jnp.float32)
        acc_ref[:, :, :] = partial.astype(jnp.bfloat16).reshape(N_DEV, chunk, h)

        rdmas = []
        for o in range(1, N_DEV):
            q = lax.rem(my_pos + o, N_DEV)
            rdma = pltpu.make_async_remote_copy(
                src_ref=acc_ref.at[q],
                dst_ref=comm_ref.at[o - 1],
                send_sem=send_sems.at[o - 1],
                recv_sem=recv_sems.at[o - 1],
                device_id=(q,),
                device_id_type=pl.DeviceIdType.MESH,
            )
            rdma.start()
            rdmas.append(rdma)

        x_mine = x_ref[pl.ds(my_pos * chunk, chunk), :]
        shared_mine = jnp.dot(x_mine.astype(jnp.bfloat16),
                              sw_ref[:, :].astype(jnp.bfloat16),
                              preferred_element_type=jnp.float32)
        for rdma in rdmas:
            rdma.wait_recv()
        out_ref[:, :] = (acc_ref[my_pos].astype(jnp.float32) + shared_mine
                         + comm_ref[0].astype(jnp.float32)
                         + comm_ref[1].astype(jnp.float32)
                         + comm_ref[2].astype(jnp.float32))
        for rdma in rdmas:
            rdma.wait_send()

    return pl.pallas_call(
        body,
        out_shape=jax.ShapeDtypeStruct((chunk, h), jnp.float32),
        in_specs=[pl.BlockSpec(memory_space=pltpu.VMEM)] * 5,
        out_specs=pl.BlockSpec(memory_space=pltpu.VMEM),
        scratch_shapes=[
            pltpu.VMEM((N_DEV, chunk, h), jnp.bfloat16),
            pltpu.VMEM((N_DEV - 1, chunk, h), jnp.bfloat16),
            pltpu.SemaphoreType.DMA((N_DEV - 1,)),
            pltpu.SemaphoreType.DMA((N_DEV - 1,)),
        ],
        compiler_params=pltpu.CompilerParams(collective_id=0),
    )(x, router_W, route_idx, expert_W, shared_W)
